# baseline (speedup 1.0000x reference)
import functools

import jax
import jax.numpy as jnp
from jax import lax
from jax.experimental import pallas as pl
from jax.experimental.pallas import tpu as pltpu

N = 4096
D = 512
ROWS_PER_BLK = 128
NUM_DEG_BLKS = N // ROWS_PER_BLK
XBLK = 512
NUM_XBLKS = N // XBLK


def _deg_body(m_ref, out_ref):
    ones = jnp.ones((N, 8), dtype=jnp.float32)
    out_ref[0] = jnp.dot(m_ref[...], ones,
                         preferred_element_type=jnp.float32)


def _deg_tc(matrix):
    return pl.pallas_call(
        _deg_body,
        grid=(NUM_DEG_BLKS,),
        in_specs=[pl.BlockSpec((ROWS_PER_BLK, N), lambda i: (i, 0))],
        out_specs=pl.BlockSpec((1, ROWS_PER_BLK, 8), lambda i: (i, 0, 0)),
        out_shape=jax.ShapeDtypeStruct((NUM_DEG_BLKS, ROWS_PER_BLK, 8),
                                       jnp.float32),
    )(matrix)


def _main_body(spref, deg_ref, mrow_ref, x_ref, lmT_ref, x3i_ref, xrow_ref,
               W1_ref, b1_ref, W2_ref, b2_ref, Wp_ref, bp_ref, out_ref,
               A_acc, s_acc, S_acc):
    i = pl.program_id(0)

    @pl.when(i == 0)
    def _init():
        A_acc[...] = jnp.zeros_like(A_acc)
        s_acc[...] = jnp.zeros_like(s_acc)
        S_acc[0, 0] = 0.0

    row = mrow_ref[0]
    nb = row != 0
    degb = deg_ref[0]
    wt = jnp.where(nb, lax.rsqrt(jnp.where(nb, degb, 1.0)), 0.0)
    lwT = lmT_ref[...] * wt
    A_acc[...] += jnp.dot(lwT, x_ref[...], preferred_element_type=jnp.float32)
    s_acc[...] += jnp.broadcast_to(
        jnp.sum(lwT, axis=1, keepdims=True), s_acc.shape)
    S_acc[0, 0] += jnp.sum(row)

    @pl.when(i == NUM_XBLKS - 1)
    def _final():
        S = S_acc[0, 0]
        rs = jnp.where(S > 0, lax.rsqrt(S), 0.0)
        flagv = spref[1]
        Wsel = jnp.where(flagv == 1, W1_ref[...], W2_ref[...])
        bsel = jnp.where(flagv == 1, b1_ref[...], b2_ref[...])
        A = A_acc[...] * rs
        sc = s_acc[:, 0:1] * rs
        ta = jnp.dot(A, Wsel, preferred_element_type=jnp.float32) + sc * bsel
        zi = jnp.dot(xrow_ref[0], Wsel,
                     preferred_element_type=jnp.float32) + bsel
        h8 = jnp.concatenate(
            [jnp.maximum(zi, 0.0), jnp.maximum(ta, 0.0)[0:7, :]], axis=0)
        p8 = jnp.maximum(x3i_ref[0], h8)
        acc = bp_ref[...]
        for m in range(8):
            acc = acc + jnp.dot(p8[m:m + 1, :], Wp_ref[m * 64:(m + 1) * 64, :],
                                preferred_element_type=jnp.float32)
        out_ref[...] = acc


def _main_tc(spref, deg8, matrix, x, lmT8, x3, W1, b1, W2, b2, Wp, bp):
    grid_spec = pltpu.PrefetchScalarGridSpec(
        num_scalar_prefetch=1,
        grid=(NUM_XBLKS,),
        in_specs=[
            pl.BlockSpec((1, 1, XBLK), lambda i, s: (i, 0, 0)),
            pl.BlockSpec((1, 1, XBLK),
                         lambda i, s: (s[0] * NUM_XBLKS + i, 0, 0)),
            pl.BlockSpec((XBLK, D), lambda i, s: (i, 0)),
            pl.BlockSpec((8, XBLK), lambda i, s: (0, i)),
            pl.BlockSpec((1, 8, 64), lambda i, s: (s[0], 0, 0)),
            pl.BlockSpec((1, 1, D), lambda i, s: (s[0], 0, 0)),
            pl.BlockSpec((D, 64), lambda i, s: (0, 0)),
            pl.BlockSpec((1, 64), lambda i, s: (0, 0)),
            pl.BlockSpec((D, 64), lambda i, s: (0, 0)),
            pl.BlockSpec((1, 64), lambda i, s: (0, 0)),
            pl.BlockSpec((D, 7), lambda i, s: (0, 0)),
            pl.BlockSpec((1, 7), lambda i, s: (0, 0)),
        ],
        out_specs=pl.BlockSpec((1, 7), lambda i, s: (0, 0)),
        scratch_shapes=[
            pltpu.VMEM((8, D), jnp.float32),
            pltpu.VMEM((8, 128), jnp.float32),
            pltpu.SMEM((1, 1), jnp.float32),
        ],
    )
    return pl.pallas_call(
        _main_body,
        grid_spec=grid_spec,
        out_shape=jax.ShapeDtypeStruct((1, 7), jnp.float32),
    )(spref, deg8, matrix.reshape(N * NUM_XBLKS, 1, XBLK), x, lmT8, x3,
      x.reshape(N, 1, D), W1, b1, W2, b2, Wp, bp)


def kernel(flag, index, matrix, x_features, x_labels, W1, b1, W2, b2, Wp, bp):
    deg = _deg_tc(matrix)[:, :, 0].reshape(NUM_XBLKS, 1, XBLK)
    spref = jnp.array([index, flag]).astype(jnp.int32)
    lmT = (x_labels != 0).astype(jnp.float32).T
    lmT8 = jnp.concatenate(
        [lmT, jnp.zeros((1, N), jnp.float32)], axis=0)
    x3 = x_features.reshape(N, 8, 64)
    out = _main_tc(spref, deg, matrix, x_features, lmT8, x3,
                   W1, b1.reshape(1, 64), W2, b2.reshape(1, 64),
                   Wp, bp.reshape(1, 7))
    return out


# speedup vs baseline: 5.4794x; 5.4794x over previous
import jax
import jax.numpy as jnp
from jax import lax
from jax.experimental import pallas as pl
from jax.experimental.pallas import tpu as pltpu

N = 4096
D = 512
ROWS_PER_BLK = 128
NUM_DEG_BLKS = N // ROWS_PER_BLK
XBLK = 512
NUM_XBLKS = N // XBLK


def _deg_body(spref, m_ref, deg_ref, row_ref):
    i = pl.program_id(0)
    mb = m_ref[...]
    ones = jnp.ones((1, N), dtype=jnp.float32)
    deg_ref[...] = lax.dot_general(
        ones, mb, (((1,), (1,)), ((), ())),
        preferred_element_type=jnp.float32)

    @pl.when(i == 0)
    def _init():
        row_ref[...] = jnp.zeros_like(row_ref)

    rel = spref[0] - i * ROWS_PER_BLK
    sel = (lax.broadcasted_iota(jnp.int32, (1, ROWS_PER_BLK), 1)
           == rel).astype(jnp.float32)
    row_ref[...] += jnp.dot(sel, mb, preferred_element_type=jnp.float32)


def _deg_tc(spref, matrix):
    grid_spec = pltpu.PrefetchScalarGridSpec(
        num_scalar_prefetch=1,
        grid=(NUM_DEG_BLKS,),
        in_specs=[pl.BlockSpec((ROWS_PER_BLK, N), lambda i, s: (i, 0))],
        out_specs=[
            pl.BlockSpec((1, ROWS_PER_BLK), lambda i, s: (0, i)),
            pl.BlockSpec((1, N), lambda i, s: (0, 0)),
        ],
    )
    return pl.pallas_call(
        _deg_body,
        grid_spec=grid_spec,
        out_shape=[jax.ShapeDtypeStruct((1, N), jnp.float32),
                   jax.ShapeDtypeStruct((1, N), jnp.float32)],
    )(spref, matrix)


def _main_body(spref, deg_ref, row_ref, x_ref, lmT_ref,
               W1_ref, b1_ref, W2_ref, b2_ref, Wp_ref, bp_ref, out_ref,
               A_acc, s_acc, xi_acc, S_acc):
    i = pl.program_id(0)

    @pl.when(i == 0)
    def _init():
        A_acc[...] = jnp.zeros_like(A_acc)
        s_acc[...] = jnp.zeros_like(s_acc)
        xi_acc[...] = jnp.zeros_like(xi_acc)
        S_acc[0, 0] = 0.0

    row = row_ref[...]
    nb = row != 0
    wt = jnp.where(nb, lax.rsqrt(jnp.where(nb, deg_ref[...], 1.0)), 0.0)
    lwT = lmT_ref[...] * wt
    xb = x_ref[...]
    A_acc[...] += jnp.dot(lwT, xb, preferred_element_type=jnp.float32)
    s_acc[...] += jnp.broadcast_to(
        jnp.sum(lwT, axis=1, keepdims=True), s_acc.shape)
    S_acc[0, 0] += jnp.sum(row)
    rel = spref[0] - i * XBLK
    sel = (lax.broadcasted_iota(jnp.int32, (1, XBLK), 1)
           == rel).astype(jnp.float32)
    xi_acc[...] += jnp.dot(sel, xb, preferred_element_type=jnp.float32)

    @pl.when(i == NUM_XBLKS - 1)
    def _final():
        S = S_acc[0, 0]
        rs = jnp.where(S > 0, lax.rsqrt(S), 0.0)
        flagv = spref[1]
        Wsel = jnp.where(flagv == 1, W1_ref[...], W2_ref[...])
        bsel = jnp.where(flagv == 1, b1_ref[...], b2_ref[...])
        A = A_acc[...] * rs
        SB = (s_acc[:, 0:1] * rs) * bsel
        ta = jnp.maximum(
            jnp.dot(A, Wsel, preferred_element_type=jnp.float32) + SB, 0.0)
        XI = xi_acc[...]
        zi = jnp.maximum(
            jnp.dot(XI, Wsel, preferred_element_type=jnp.float32) + bsel, 0.0)
        h = jnp.concatenate(
            [zi] + [ta[l:l + 1, :] for l in range(7)], axis=1)
        P = jnp.maximum(XI, h)
        out_ref[...] = (jnp.dot(P, Wp_ref[...],
                                preferred_element_type=jnp.float32)
                        + bp_ref[...])


def _main_tc(spref, deg_row, mrow, x, lmT8, W1, b1, W2, b2, Wp, bp):
    grid_spec = pltpu.PrefetchScalarGridSpec(
        num_scalar_prefetch=1,
        grid=(NUM_XBLKS,),
        in_specs=[
            pl.BlockSpec((1, XBLK), lambda i, s: (0, i)),
            pl.BlockSpec((1, XBLK), lambda i, s: (0, i)),
            pl.BlockSpec((XBLK, D), lambda i, s: (i, 0)),
            pl.BlockSpec((8, XBLK), lambda i, s: (0, i)),
            pl.BlockSpec((D, 64), lambda i, s: (0, 0)),
            pl.BlockSpec((1, 64), lambda i, s: (0, 0)),
            pl.BlockSpec((D, 64), lambda i, s: (0, 0)),
            pl.BlockSpec((1, 64), lambda i, s: (0, 0)),
            pl.BlockSpec((D, 7), lambda i, s: (0, 0)),
            pl.BlockSpec((1, 7), lambda i, s: (0, 0)),
        ],
        out_specs=pl.BlockSpec((1, 7), lambda i, s: (0, 0)),
        scratch_shapes=[
            pltpu.VMEM((8, D), jnp.float32),
            pltpu.VMEM((8, 128), jnp.float32),
            pltpu.VMEM((1, D), jnp.float32),
            pltpu.SMEM((1, 1), jnp.float32),
        ],
    )
    return pl.pallas_call(
        _main_body,
        grid_spec=grid_spec,
        out_shape=jax.ShapeDtypeStruct((1, 7), jnp.float32),
    )(spref, deg_row, mrow, x, lmT8, W1, b1, W2, b2, Wp, bp)


def kernel(flag, index, matrix, x_features, x_labels, W1, b1, W2, b2, Wp, bp):
    spref = jnp.array([index, flag]).astype(jnp.int32)
    deg_row, mrow = _deg_tc(spref, matrix)
    lmT = (x_labels != 0).astype(jnp.float32).T
    lmT8 = jnp.concatenate(
        [lmT, jnp.zeros((1, N), jnp.float32)], axis=0)
    out = _main_tc(spref, deg_row, mrow, x_features, lmT8,
                   W1, b1.reshape(1, 64), W2, b2.reshape(1, 64),
                   Wp, bp.reshape(1, 7))
    return out
